# Initial kernel scaffold; baseline (speedup 1.0000x reference)
import jax
import jax.numpy as jnp
from jax.experimental import pallas as pl


def kernel(x, edge_index, W1, b1, W2, b2):
    raise NotImplementedError("write your pallas kernel here")


# speedup vs baseline: 35.9227x; 35.9227x over previous
import jax
import jax.numpy as jnp
from jax import lax
from jax.experimental import pallas as pl
from jax.experimental.pallas import tpu as pltpu
from jax.experimental.pallas import tpu_sc as plsc

N_NODES = 100000
N_EDGES = 3200000
NPAD = 102400
NODES_PER_TILE = NPAD // 16
NODES_PER_WORKER = NPAD // 32
CHUNK = 80
SUB = 16
SPLITS = CHUNK // SUB
ROWS = N_EDGES // CHUNK
ROWS_PER_SC = ROWS // 2
ROWS_PER_TILE = ROWS_PER_SC // 16
DEG_ROWS_PER_TILE = ROWS // 16
STAGE = 10
NCHUNK = 800
F = 5
FP = 8
H = 16


def _mesh():
    return plsc.VectorSubcoreMesh(core_axis_name="c", subcore_axis_name="s")


_CPARAMS = pltpu.CompilerParams(needs_layout_passes=False,
                                use_tc_tiling_on_sc=False)


def _rsqrt_nr(d):
    i = lax.bitcast_convert_type(d, jnp.int32)
    i = jnp.int32(0x5F3759DF) - lax.shift_right_arithmetic(i, 1)
    y = lax.bitcast_convert_type(i, jnp.float32)
    for _ in range(3):
        y = y * (1.5 - 0.5 * d * y * y)
    return y


def _kernel_a(x_pad, src2d, dst3d, zeros8, zeros1, ones1,
              acc_part, dis_out,
              deg_s, u_s, acc_s,
              degv, xv, uv, sbuf, dbuf, dbuf2, rows_v, ones_v, sem):
    cid = lax.axis_index("c")
    tid = lax.axis_index("s")
    nbase = tid * NODES_PER_TILE

    for i in range(FP):
        pltpu.sync_copy(
            zeros8,
            acc_s.at[pl.ds(nbase + i * (NODES_PER_TILE // FP),
                           NODES_PER_TILE // FP), :])
    pltpu.sync_copy(zeros1, deg_s.at[pl.ds(nbase, NODES_PER_TILE)])
    pltpu.sync_copy(ones1, ones_v)
    plsc.subcore_barrier()

    ones16 = ones_v.at[pl.ds(0, SUB)]

    def deg_step(s, _):
        row0 = tid * DEG_ROWS_PER_TILE + s * STAGE
        pltpu.sync_copy(dst3d.at[pl.ds(row0, STAGE), :, :], dbuf2)
        for j in range(STAGE):
            for k in range(SPLITS):
                pltpu.sync_copy(ones16, deg_s.at[dbuf2.at[j, k]], add=True)
        return 0

    lax.fori_loop(0, DEG_ROWS_PER_TILE // STAGE, deg_step, 0)
    plsc.subcore_barrier()

    def node_chunk(c, _):
        coff = nbase + c * NCHUNK
        pltpu.sync_copy(deg_s.at[pl.ds(coff, NCHUNK)], degv)
        pltpu.sync_copy(x_pad.at[pl.ds(coff, NCHUNK), :], xv)
        pltpu.sync_copy(zeros8, uv)

        def node_step(jb, _):
            off = jb * 16
            r = _rsqrt_nr(degv[pl.ds(off, 16)] + 1.0)
            degv[pl.ds(off, 16)] = r
            nidx = lax.iota(jnp.int32, 16) + off
            for k in range(F):
                kv = jnp.full((16,), k, dtype=jnp.int32)
                xk = plsc.load_gather(xv, [nidx, kv])
                plsc.store_scatter(uv, [nidx, kv], r * xk)
            return 0

        lax.fori_loop(0, NCHUNK // 16, node_step, 0)
        pltpu.sync_copy(uv, u_s.at[pl.ds(coff, NCHUNK), :])

        @pl.when(cid == 0)
        def _():
            pltpu.sync_copy(degv, dis_out.at[pl.ds(coff, NCHUNK)])

        return 0

    lax.fori_loop(0, NODES_PER_TILE // NCHUNK, node_chunk, 0)
    plsc.subcore_barrier()

    def edge_step(s, _):
        row0 = cid * ROWS_PER_SC + tid * ROWS_PER_TILE + s * STAGE
        pltpu.sync_copy(src2d.at[pl.ds(row0, STAGE), :], sbuf)
        pltpu.sync_copy(dst3d.at[pl.ds(row0, STAGE), :, :], dbuf)
        for j in range(STAGE):
            pltpu.async_copy(u_s.at[sbuf.at[j]], rows_v, sem).wait()
            for k in range(SPLITS):
                pltpu.sync_copy(rows_v.at[pl.ds(k * SUB, SUB), :],
                                acc_s.at[dbuf.at[j, k]], add=True)
        return 0

    lax.fori_loop(0, ROWS_PER_TILE // STAGE, edge_step, 0)
    plsc.subcore_barrier()

    pltpu.sync_copy(acc_s.at[pl.ds(nbase, NODES_PER_TILE), :],
                    acc_part.at[pl.ds(cid * NPAD + nbase, NODES_PER_TILE), :])


def _kernel_b(acc_part, dis, x_pad, w1, b1, w2,
              v_out,
              a0v, a1v, xv, disv, vv, w1v, b1v, w2v):
    cid = lax.axis_index("c")
    tid = lax.axis_index("s")
    wid = cid * 16 + tid
    nbase = wid * NODES_PER_WORKER

    pltpu.sync_copy(acc_part.at[pl.ds(nbase, NODES_PER_WORKER), :], a0v)
    pltpu.sync_copy(acc_part.at[pl.ds(NPAD + nbase, NODES_PER_WORKER), :], a1v)
    pltpu.sync_copy(x_pad.at[pl.ds(nbase, NODES_PER_WORKER), :], xv)
    pltpu.sync_copy(dis.at[pl.ds(nbase, NODES_PER_WORKER)], disv)
    pltpu.sync_copy(w1, w1v)
    pltpu.sync_copy(b1, b1v)
    pltpu.sync_copy(w2, w2v)

    w1rows = [w1v[k, :] for k in range(F)]
    b1vec = b1v[:]
    w2vec = w2v[:]

    def block(jb, _):
        off = jb * 16
        nidx = lax.iota(jnp.int32, 16) + off
        r = disv[pl.ds(off, 16)]
        ys = []
        for k in range(F):
            kv = jnp.full((16,), k, dtype=jnp.int32)
            a0 = plsc.load_gather(a0v, [nidx, kv])
            a1 = plsc.load_gather(a1v, [nidx, kv])
            xk = plsc.load_gather(xv, [nidx, kv])
            ys.append(r * (a0 + a1 + r * xk))
        z = jnp.zeros((16,), jnp.float32)
        for j in range(H):
            acc = jnp.broadcast_to(b1vec[j], (16,))
            for k in range(F):
                acc = acc + ys[k] * w1rows[k][j]
            h = jnp.maximum(acc, 0.0)
            z = z + h * w2vec[j]
        vv[pl.ds(off, 16)] = r * z
        return 0

    lax.fori_loop(0, NODES_PER_WORKER // 16, block, 0)
    pltpu.sync_copy(vv, v_out.at[pl.ds(nbase, NODES_PER_WORKER)])


def _kernel_c(v, src2d, dst3d, zeros1,
              acc2_part,
              v_s, acc2_s,
              sbuf, dbuf, val_v, sem):
    cid = lax.axis_index("c")
    tid = lax.axis_index("s")
    nbase = tid * NODES_PER_TILE

    pltpu.sync_copy(v.at[pl.ds(nbase, NODES_PER_TILE)],
                    v_s.at[pl.ds(nbase, NODES_PER_TILE)])
    pltpu.sync_copy(zeros1, acc2_s.at[pl.ds(nbase, NODES_PER_TILE)])
    plsc.subcore_barrier()

    def edge_step(s, _):
        row0 = cid * ROWS_PER_SC + tid * ROWS_PER_TILE + s * STAGE
        pltpu.sync_copy(src2d.at[pl.ds(row0, STAGE), :], sbuf)
        pltpu.sync_copy(dst3d.at[pl.ds(row0, STAGE), :, :], dbuf)
        for j in range(STAGE):
            pltpu.async_copy(v_s.at[sbuf.at[j]], val_v, sem).wait()
            for k in range(SPLITS):
                pltpu.sync_copy(val_v.at[pl.ds(k * SUB, SUB)],
                                acc2_s.at[dbuf.at[j, k]], add=True)
        return 0

    lax.fori_loop(0, ROWS_PER_TILE // STAGE, edge_step, 0)
    plsc.subcore_barrier()

    pltpu.sync_copy(acc2_s.at[pl.ds(nbase, NODES_PER_TILE)],
                    acc2_part.at[pl.ds(cid * NPAD + nbase, NODES_PER_TILE)])


def _kernel_d(acc2_part, v, dis, b2pad,
              out_pad,
              a0v, a1v, vv, disv, ov, b2v):
    cid = lax.axis_index("c")
    tid = lax.axis_index("s")
    wid = cid * 16 + tid
    nbase = wid * NODES_PER_WORKER

    pltpu.sync_copy(acc2_part.at[pl.ds(nbase, NODES_PER_WORKER)], a0v)
    pltpu.sync_copy(acc2_part.at[pl.ds(NPAD + nbase, NODES_PER_WORKER)], a1v)
    pltpu.sync_copy(v.at[pl.ds(nbase, NODES_PER_WORKER)], vv)
    pltpu.sync_copy(dis.at[pl.ds(nbase, NODES_PER_WORKER)], disv)
    pltpu.sync_copy(b2pad, b2v)
    b2s = b2v[:][0]

    def block(jb, _):
        off = jb * 16
        r = disv[pl.ds(off, 16)]
        ov[pl.ds(off, 16)] = (
            r * (a0v[pl.ds(off, 16)] + a1v[pl.ds(off, 16)]
                 + vv[pl.ds(off, 16)]) + b2s)
        return 0

    lax.fori_loop(0, NODES_PER_WORKER // 16, block, 0)
    pltpu.sync_copy(ov, out_pad.at[pl.ds(nbase, NODES_PER_WORKER)])


def kernel(x, edge_index, W1, b1, W2, b2):
    src2d = edge_index[0].astype(jnp.int32).reshape(ROWS, CHUNK)
    dst3d = edge_index[1].astype(jnp.int32).reshape(ROWS, SPLITS, SUB)
    x_pad = jnp.pad(x, ((0, NPAD - N_NODES), (0, 0)))
    zcol = x[:NODES_PER_TILE, 0] * 0.0
    zeros8 = jnp.broadcast_to(zcol[:NODES_PER_TILE // FP, None],
                              (NODES_PER_TILE // FP, FP))
    zeros1 = zcol
    ones1 = zcol[:CHUNK] + 1.0
    w2r = W2.reshape(H)
    b2pad = jnp.broadcast_to(b2, (16,))

    f32 = jnp.float32
    i32 = jnp.int32

    ka = pl.kernel(
        _kernel_a,
        out_type=(jax.ShapeDtypeStruct((2 * NPAD, FP), f32),
                  jax.ShapeDtypeStruct((NPAD,), f32)),
        mesh=_mesh(),
        compiler_params=_CPARAMS,
        scratch_types=(
            pltpu.VMEM_SHARED((NPAD,), f32),
            pltpu.VMEM_SHARED((NPAD, FP), f32),
            pltpu.VMEM_SHARED((NPAD, FP), f32),
            pltpu.VMEM((NCHUNK,), f32),
            pltpu.VMEM((NCHUNK, F), f32),
            pltpu.VMEM((NCHUNK, FP), f32),
            pltpu.VMEM((STAGE, CHUNK), i32),
            pltpu.VMEM((STAGE, SPLITS, SUB), i32),
            pltpu.VMEM((STAGE, SPLITS, SUB), i32),
            pltpu.VMEM((CHUNK, FP), f32),
            pltpu.VMEM((CHUNK,), f32),
            pltpu.SemaphoreType.DMA,
        ))
    acc_part, dis = ka(x_pad, src2d, dst3d, zeros8, zeros1, ones1)

    kb = pl.kernel(
        _kernel_b,
        out_type=jax.ShapeDtypeStruct((NPAD,), f32),
        mesh=_mesh(),
        compiler_params=_CPARAMS,
        scratch_types=(
            pltpu.VMEM((NODES_PER_WORKER, FP), f32),
            pltpu.VMEM((NODES_PER_WORKER, FP), f32),
            pltpu.VMEM((NODES_PER_WORKER, F), f32),
            pltpu.VMEM((NODES_PER_WORKER,), f32),
            pltpu.VMEM((NODES_PER_WORKER,), f32),
            pltpu.VMEM((F, H), f32),
            pltpu.VMEM((H,), f32),
            pltpu.VMEM((H,), f32),
        ))
    v = kb(acc_part, dis, x_pad, W1, b1, w2r)

    kc = pl.kernel(
        _kernel_c,
        out_type=jax.ShapeDtypeStruct((2 * NPAD,), f32),
        mesh=_mesh(),
        compiler_params=_CPARAMS,
        scratch_types=(
            pltpu.VMEM_SHARED((NPAD,), f32),
            pltpu.VMEM_SHARED((NPAD,), f32),
            pltpu.VMEM((STAGE, CHUNK), i32),
            pltpu.VMEM((STAGE, SPLITS, SUB), i32),
            pltpu.VMEM((CHUNK,), f32),
            pltpu.SemaphoreType.DMA,
        ))
    acc2_part = kc(v, src2d, dst3d, zeros1)

    kd = pl.kernel(
        _kernel_d,
        out_type=jax.ShapeDtypeStruct((NPAD,), f32),
        mesh=_mesh(),
        compiler_params=_CPARAMS,
        scratch_types=(
            pltpu.VMEM((NODES_PER_WORKER,), f32),
            pltpu.VMEM((NODES_PER_WORKER,), f32),
            pltpu.VMEM((NODES_PER_WORKER,), f32),
            pltpu.VMEM((NODES_PER_WORKER,), f32),
            pltpu.VMEM((NODES_PER_WORKER,), f32),
            pltpu.VMEM((16,), f32),
        ))
    out_pad = kd(acc2_part, v, dis, b2pad)

    return out_pad[:N_NODES].reshape(N_NODES, 1)


# speedup vs baseline: 55.1298x; 1.5347x over previous
import jax
import jax.numpy as jnp
from jax import lax
from jax.experimental import pallas as pl
from jax.experimental.pallas import tpu as pltpu
from jax.experimental.pallas import tpu_sc as plsc

N_NODES = 100000
N_EDGES = 3200000
NPAD = 102400
NODES_PER_TILE = NPAD // 16
NODES_PER_WORKER = NPAD // 32
CHUNK = 80
SUB = 40
SPLITS = CHUNK // SUB
ROWS = N_EDGES // CHUNK
ROWS_PER_SC = ROWS // 2
ROWS_PER_TILE = ROWS_PER_SC // 16
DEG_ROWS_PER_TILE = ROWS // 16
STAGE = 10
NCHUNK = 800
F = 5
FP = 8
H = 16


def _mesh():
    return plsc.VectorSubcoreMesh(core_axis_name="c", subcore_axis_name="s")


_CPARAMS = pltpu.CompilerParams(needs_layout_passes=False,
                                use_tc_tiling_on_sc=False)


def _rsqrt_nr(d):
    i = lax.bitcast_convert_type(d, jnp.int32)
    i = jnp.int32(0x5F3759DF) - lax.shift_right_arithmetic(i, 1)
    y = lax.bitcast_convert_type(i, jnp.float32)
    for _ in range(3):
        y = y * (1.5 - 0.5 * d * y * y)
    return y


def _kernel_a(x_pad, src2d, dst3d, zeros8, zeros1, ones1,
              acc_part, dis_out,
              deg_s, u_s, acc_s,
              degv, xv, uv, sbuf, dbuf, dbuf2, rows_v, ones_v, sem):
    cid = lax.axis_index("c")
    tid = lax.axis_index("s")
    nbase = tid * NODES_PER_TILE

    for i in range(FP):
        pltpu.sync_copy(
            zeros8,
            acc_s.at[pl.ds(nbase + i * (NODES_PER_TILE // FP),
                           NODES_PER_TILE // FP), :])
    pltpu.sync_copy(zeros1, deg_s.at[pl.ds(nbase, NODES_PER_TILE)])
    pltpu.sync_copy(ones1, ones_v)
    plsc.subcore_barrier()

    ones_sub = ones_v.at[pl.ds(0, SUB)]

    def deg_step(s, _):
        row0 = tid * DEG_ROWS_PER_TILE + s * STAGE
        pltpu.sync_copy(dst3d.at[pl.ds(row0, STAGE), :, :], dbuf2)
        for j in range(STAGE):
            for k in range(SPLITS):
                pltpu.sync_copy(ones_sub, deg_s.at[dbuf2.at[j, k]], add=True)
        return 0

    lax.fori_loop(0, DEG_ROWS_PER_TILE // STAGE, deg_step, 0)
    plsc.subcore_barrier()

    def node_chunk(c, _):
        coff = nbase + c * NCHUNK
        pltpu.sync_copy(deg_s.at[pl.ds(coff, NCHUNK)], degv)
        pltpu.sync_copy(x_pad.at[pl.ds(coff, NCHUNK), :], xv)
        pltpu.sync_copy(zeros8, uv)

        def node_step(jb, _):
            off = jb * 16
            r = _rsqrt_nr(degv[pl.ds(off, 16)] + 1.0)
            degv[pl.ds(off, 16)] = r
            nidx = lax.iota(jnp.int32, 16) + off
            for k in range(F):
                kv = jnp.full((16,), k, dtype=jnp.int32)
                xk = plsc.load_gather(xv, [nidx, kv])
                plsc.store_scatter(uv, [nidx, kv], r * xk)
            return 0

        lax.fori_loop(0, NCHUNK // 16, node_step, 0)
        pltpu.sync_copy(uv, u_s.at[pl.ds(coff, NCHUNK), :])

        @pl.when(cid == 0)
        def _():
            pltpu.sync_copy(degv, dis_out.at[pl.ds(coff, NCHUNK)])

        return 0

    lax.fori_loop(0, NODES_PER_TILE // NCHUNK, node_chunk, 0)
    plsc.subcore_barrier()

    def edge_step(s, _):
        row0 = cid * ROWS_PER_SC + tid * ROWS_PER_TILE + s * STAGE
        pltpu.sync_copy(src2d.at[pl.ds(row0, STAGE), :], sbuf)
        pltpu.sync_copy(dst3d.at[pl.ds(row0, STAGE), :, :], dbuf)
        for j in range(STAGE):
            pltpu.async_copy(u_s.at[sbuf.at[j]], rows_v, sem).wait()
            for k in range(SPLITS):
                pltpu.sync_copy(rows_v.at[pl.ds(k * SUB, SUB), :],
                                acc_s.at[dbuf.at[j, k]], add=True)
        return 0

    lax.fori_loop(0, ROWS_PER_TILE // STAGE, edge_step, 0)
    plsc.subcore_barrier()

    pltpu.sync_copy(acc_s.at[pl.ds(nbase, NODES_PER_TILE), :],
                    acc_part.at[pl.ds(cid * NPAD + nbase, NODES_PER_TILE), :])


def _kernel_b(acc_part, dis, x_pad, w1, b1, w2,
              v_out,
              a0v, a1v, xv, disv, vv, w1v, b1v, w2v):
    cid = lax.axis_index("c")
    tid = lax.axis_index("s")
    wid = cid * 16 + tid
    nbase = wid * NODES_PER_WORKER

    pltpu.sync_copy(acc_part.at[pl.ds(nbase, NODES_PER_WORKER), :], a0v)
    pltpu.sync_copy(acc_part.at[pl.ds(NPAD + nbase, NODES_PER_WORKER), :], a1v)
    pltpu.sync_copy(x_pad.at[pl.ds(nbase, NODES_PER_WORKER), :], xv)
    pltpu.sync_copy(dis.at[pl.ds(nbase, NODES_PER_WORKER)], disv)
    pltpu.sync_copy(w1, w1v)
    pltpu.sync_copy(b1, b1v)
    pltpu.sync_copy(w2, w2v)

    w1rows = [w1v[k, :] for k in range(F)]
    b1vec = b1v[:]
    w2vec = w2v[:]

    def block(jb, _):
        off = jb * 16
        nidx = lax.iota(jnp.int32, 16) + off
        r = disv[pl.ds(off, 16)]
        ys = []
        for k in range(F):
            kv = jnp.full((16,), k, dtype=jnp.int32)
            a0 = plsc.load_gather(a0v, [nidx, kv])
            a1 = plsc.load_gather(a1v, [nidx, kv])
            xk = plsc.load_gather(xv, [nidx, kv])
            ys.append(r * (a0 + a1 + r * xk))
        z = jnp.zeros((16,), jnp.float32)
        for j in range(H):
            acc = jnp.broadcast_to(b1vec[j], (16,))
            for k in range(F):
                acc = acc + ys[k] * w1rows[k][j]
            h = jnp.maximum(acc, 0.0)
            z = z + h * w2vec[j]
        vv[pl.ds(off, 16)] = r * z
        return 0

    lax.fori_loop(0, NODES_PER_WORKER // 16, block, 0)
    pltpu.sync_copy(vv, v_out.at[pl.ds(nbase, NODES_PER_WORKER)])


def _kernel_c(v, src2d, dst3d, zeros1,
              acc2_part,
              v_s, acc2_s,
              sbuf, dbuf, val_v, sem):
    cid = lax.axis_index("c")
    tid = lax.axis_index("s")
    nbase = tid * NODES_PER_TILE

    pltpu.sync_copy(v.at[pl.ds(nbase, NODES_PER_TILE)],
                    v_s.at[pl.ds(nbase, NODES_PER_TILE)])
    pltpu.sync_copy(zeros1, acc2_s.at[pl.ds(nbase, NODES_PER_TILE)])
    plsc.subcore_barrier()

    def edge_step(s, _):
        row0 = cid * ROWS_PER_SC + tid * ROWS_PER_TILE + s * STAGE
        pltpu.sync_copy(src2d.at[pl.ds(row0, STAGE), :], sbuf)
        pltpu.sync_copy(dst3d.at[pl.ds(row0, STAGE), :, :], dbuf)
        for j in range(STAGE):
            pltpu.async_copy(v_s.at[sbuf.at[j]], val_v, sem).wait()
            for k in range(SPLITS):
                pltpu.sync_copy(val_v.at[pl.ds(k * SUB, SUB)],
                                acc2_s.at[dbuf.at[j, k]], add=True)
        return 0

    lax.fori_loop(0, ROWS_PER_TILE // STAGE, edge_step, 0)
    plsc.subcore_barrier()

    pltpu.sync_copy(acc2_s.at[pl.ds(nbase, NODES_PER_TILE)],
                    acc2_part.at[pl.ds(cid * NPAD + nbase, NODES_PER_TILE)])


def _kernel_d(acc2_part, v, dis, b2pad,
              out_pad,
              a0v, a1v, vv, disv, ov, b2v):
    cid = lax.axis_index("c")
    tid = lax.axis_index("s")
    wid = cid * 16 + tid
    nbase = wid * NODES_PER_WORKER

    pltpu.sync_copy(acc2_part.at[pl.ds(nbase, NODES_PER_WORKER)], a0v)
    pltpu.sync_copy(acc2_part.at[pl.ds(NPAD + nbase, NODES_PER_WORKER)], a1v)
    pltpu.sync_copy(v.at[pl.ds(nbase, NODES_PER_WORKER)], vv)
    pltpu.sync_copy(dis.at[pl.ds(nbase, NODES_PER_WORKER)], disv)
    pltpu.sync_copy(b2pad, b2v)
    b2s = b2v[:][0]

    def block(jb, _):
        off = jb * 16
        r = disv[pl.ds(off, 16)]
        ov[pl.ds(off, 16)] = (
            r * (a0v[pl.ds(off, 16)] + a1v[pl.ds(off, 16)]
                 + vv[pl.ds(off, 16)]) + b2s)
        return 0

    lax.fori_loop(0, NODES_PER_WORKER // 16, block, 0)
    pltpu.sync_copy(ov, out_pad.at[pl.ds(nbase, NODES_PER_WORKER)])


def kernel(x, edge_index, W1, b1, W2, b2):
    src2d = edge_index[0].astype(jnp.int32).reshape(ROWS, CHUNK)
    dst3d = edge_index[1].astype(jnp.int32).reshape(ROWS, SPLITS, SUB)
    x_pad = jnp.pad(x, ((0, NPAD - N_NODES), (0, 0)))
    zcol = x[:NODES_PER_TILE, 0] * 0.0
    zeros8 = jnp.broadcast_to(zcol[:NODES_PER_TILE // FP, None],
                              (NODES_PER_TILE // FP, FP))
    zeros1 = zcol
    ones1 = zcol[:CHUNK] + 1.0
    w2r = W2.reshape(H)
    b2pad = jnp.broadcast_to(b2, (16,))

    f32 = jnp.float32
    i32 = jnp.int32

    ka = pl.kernel(
        _kernel_a,
        out_type=(jax.ShapeDtypeStruct((2 * NPAD, FP), f32),
                  jax.ShapeDtypeStruct((NPAD,), f32)),
        mesh=_mesh(),
        compiler_params=_CPARAMS,
        scratch_types=(
            pltpu.VMEM_SHARED((NPAD,), f32),
            pltpu.VMEM_SHARED((NPAD, FP), f32),
            pltpu.VMEM_SHARED((NPAD, FP), f32),
            pltpu.VMEM((NCHUNK,), f32),
            pltpu.VMEM((NCHUNK, F), f32),
            pltpu.VMEM((NCHUNK, FP), f32),
            pltpu.VMEM((STAGE, CHUNK), i32),
            pltpu.VMEM((STAGE, SPLITS, SUB), i32),
            pltpu.VMEM((STAGE, SPLITS, SUB), i32),
            pltpu.VMEM((CHUNK, FP), f32),
            pltpu.VMEM((CHUNK,), f32),
            pltpu.SemaphoreType.DMA,
        ))
    acc_part, dis = ka(x_pad, src2d, dst3d, zeros8, zeros1, ones1)

    kb = pl.kernel(
        _kernel_b,
        out_type=jax.ShapeDtypeStruct((NPAD,), f32),
        mesh=_mesh(),
        compiler_params=_CPARAMS,
        scratch_types=(
            pltpu.VMEM((NODES_PER_WORKER, FP), f32),
            pltpu.VMEM((NODES_PER_WORKER, FP), f32),
            pltpu.VMEM((NODES_PER_WORKER, F), f32),
            pltpu.VMEM((NODES_PER_WORKER,), f32),
            pltpu.VMEM((NODES_PER_WORKER,), f32),
            pltpu.VMEM((F, H), f32),
            pltpu.VMEM((H,), f32),
            pltpu.VMEM((H,), f32),
        ))
    v = kb(acc_part, dis, x_pad, W1, b1, w2r)

    kc = pl.kernel(
        _kernel_c,
        out_type=jax.ShapeDtypeStruct((2 * NPAD,), f32),
        mesh=_mesh(),
        compiler_params=_CPARAMS,
        scratch_types=(
            pltpu.VMEM_SHARED((NPAD,), f32),
            pltpu.VMEM_SHARED((NPAD,), f32),
            pltpu.VMEM((STAGE, CHUNK), i32),
            pltpu.VMEM((STAGE, SPLITS, SUB), i32),
            pltpu.VMEM((CHUNK,), f32),
            pltpu.SemaphoreType.DMA,
        ))
    acc2_part = kc(v, src2d, dst3d, zeros1)

    kd = pl.kernel(
        _kernel_d,
        out_type=jax.ShapeDtypeStruct((NPAD,), f32),
        mesh=_mesh(),
        compiler_params=_CPARAMS,
        scratch_types=(
            pltpu.VMEM((NODES_PER_WORKER,), f32),
            pltpu.VMEM((NODES_PER_WORKER,), f32),
            pltpu.VMEM((NODES_PER_WORKER,), f32),
            pltpu.VMEM((NODES_PER_WORKER,), f32),
            pltpu.VMEM((NODES_PER_WORKER,), f32),
            pltpu.VMEM((16,), f32),
        ))
    out_pad = kd(acc2_part, v, dis, b2pad)

    return out_pad[:N_NODES].reshape(N_NODES, 1)
